# baseline (speedup 1.0000x reference)
import jax
import jax.numpy as jnp
from jax.experimental import pallas as pl
from jax.experimental.pallas import tpu as pltpu

G = 3
E = 8
GE = G * E
D = 768
F = 3072
N = 2048
TEMP = 1.0

NF = 2
F2 = F // NF
TN = 1024
NT = N // TN
CH = 512


def _moe_body(x32_ref, xb_ref, Wg_ref, bg_ref, Wr_ref, br_ref,
              w1_ref, b1_ref, w2_ref, b2all_ref, out_ref, w_sc, h_sc):
    j = pl.program_id(0)
    f = pl.program_id(1)

    @pl.when(jnp.logical_and(j == 0, f == 0))
    def _router_and_init():
        xb = xb_ref[...]
        gl = jnp.dot(xb, Wg_ref[...].astype(jnp.bfloat16),
                     preferred_element_type=jnp.float32)
        gl = (gl + bg_ref[...]) / TEMP
        gl = gl - jnp.max(gl, axis=-1, keepdims=True)
        gex = jnp.exp(gl)
        gp = gex / jnp.sum(gex, axis=-1, keepdims=True)
        parts = []
        for g in range(G):
            el = jnp.dot(xb, Wr_ref[g].astype(jnp.bfloat16),
                         preferred_element_type=jnp.float32)
            el = (el + br_ref[g:g + 1, :]) / TEMP
            el = el - jnp.max(el, axis=-1, keepdims=True)
            eex = jnp.exp(el)
            ep = eex / jnp.sum(eex, axis=-1, keepdims=True)
            parts.append(ep * gp[:, g:g + 1])
        w = jnp.concatenate(parts, axis=-1)
        w_sc[...] = w
        out_ref[...] = x32_ref[...] + jnp.dot(
            w.astype(jnp.bfloat16), b2all_ref[...].astype(jnp.bfloat16),
            preferred_element_type=jnp.float32)

    b1 = b1_ref[0].astype(jnp.bfloat16)
    lane = jax.lax.broadcasted_iota(jnp.int32, (1, GE), 1)
    for t in range(NT):
        rows = pl.ds(t * TN, TN)
        wcol = jnp.sum(jnp.where(lane == j, w_sc[rows, :], 0.0),
                       axis=-1, keepdims=True)
        xt = xb_ref[rows, :]
        for c in range(F2 // CH):
            cs = slice(c * CH, (c + 1) * CH)
            w1c = w1_ref[0, :, cs].astype(jnp.bfloat16)
            hc = jnp.dot(xt, w1c,
                         preferred_element_type=jnp.float32).astype(jnp.bfloat16)
            h_sc[:, cs] = jax.nn.gelu(hc + b1[:, cs])
        acc = None
        for c in range(F2 // CH):
            cs = slice(c * CH, (c + 1) * CH)
            w2c = w2_ref[0, cs, :].astype(jnp.bfloat16)
            p = jnp.dot(h_sc[:, cs], w2c,
                        preferred_element_type=jnp.float32)
            acc = p if acc is None else acc + p
        out_ref[rows, :] = out_ref[rows, :] + wcol * acc


def kernel(x, Wg, bg, Wr, br, W1, b1, W2, b2):
    xb = x.astype(jnp.bfloat16)
    W1r = W1.reshape(GE, D, F)
    W2r = W2.reshape(GE, F, D)
    b1r = b1.reshape(GE, 1, F)
    b2all = b2.reshape(GE, D)
    bg2 = bg.reshape(1, G)

    return pl.pallas_call(
        _moe_body,
        grid=(GE, NF),
        in_specs=[
            pl.BlockSpec((N, D), lambda j, f: (0, 0)),
            pl.BlockSpec((N, D), lambda j, f: (0, 0)),
            pl.BlockSpec((D, G), lambda j, f: (0, 0)),
            pl.BlockSpec((1, G), lambda j, f: (0, 0)),
            pl.BlockSpec((G, D, E), lambda j, f: (0, 0, 0)),
            pl.BlockSpec((G, E), lambda j, f: (0, 0)),
            pl.BlockSpec((1, D, F2), lambda j, f: (j, 0, f)),
            pl.BlockSpec((1, 1, F2), lambda j, f: (j, 0, f)),
            pl.BlockSpec((1, F2, D), lambda j, f: (j, f, 0)),
            pl.BlockSpec((GE, D), lambda j, f: (0, 0)),
        ],
        out_specs=pl.BlockSpec((N, D), lambda j, f: (0, 0)),
        out_shape=jax.ShapeDtypeStruct((N, D), jnp.float32),
        scratch_shapes=[pltpu.VMEM((N, GE), jnp.float32),
                        pltpu.VMEM((TN, F2), jnp.bfloat16)],
        compiler_params=pltpu.CompilerParams(
            dimension_semantics=("arbitrary", "arbitrary"),
        ),
    )(x, xb, Wg, bg2, Wr, br, W1r, b1r, W2r, b2all)


# speedup vs baseline: 1.0251x; 1.0251x over previous
import jax
import jax.numpy as jnp
from jax.experimental import pallas as pl
from jax.experimental.pallas import tpu as pltpu

G = 3
E = 8
GE = G * E
D = 768
F = 3072
N = 2048
TEMP = 1.0

NF = 2
F2 = F // NF
TN = 512
NT = N // TN


def _moe_body(x32_ref, xb_ref, Wg_ref, bg_ref, Wr_ref, br_ref,
              w1_ref, b1_ref, w2_ref, b2all_ref, out_ref, w_sc):
    j = pl.program_id(0)
    f = pl.program_id(1)

    @pl.when(jnp.logical_and(j == 0, f == 0))
    def _router_and_init():
        xb = xb_ref[...]
        gl = jnp.dot(xb, Wg_ref[...].astype(jnp.bfloat16),
                     preferred_element_type=jnp.float32)
        gl = (gl + bg_ref[...]) / TEMP
        gl = gl - jnp.max(gl, axis=-1, keepdims=True)
        gex = jnp.exp(gl)
        gp = gex / jnp.sum(gex, axis=-1, keepdims=True)
        parts = []
        for g in range(G):
            el = jnp.dot(xb, Wr_ref[g].astype(jnp.bfloat16),
                         preferred_element_type=jnp.float32)
            el = (el + br_ref[g:g + 1, :]) / TEMP
            el = el - jnp.max(el, axis=-1, keepdims=True)
            eex = jnp.exp(el)
            ep = eex / jnp.sum(eex, axis=-1, keepdims=True)
            parts.append(ep * gp[:, g:g + 1])
        w = jnp.concatenate(parts, axis=-1)
        w_sc[...] = w
        out_ref[...] = x32_ref[...] + jnp.dot(
            w.astype(jnp.bfloat16), b2all_ref[...].astype(jnp.bfloat16),
            preferred_element_type=jnp.float32)

    b1 = b1_ref[0].astype(jnp.bfloat16)
    lane = jax.lax.broadcasted_iota(jnp.int32, (1, GE), 1)
    for t in range(NT):
        rows = pl.ds(t * TN, TN)
        wcol = jnp.sum(jnp.where(lane == j, w_sc[rows, :], 0.0),
                       axis=-1, keepdims=True)
        xt = xb_ref[rows, :]
        h = jnp.dot(xt, w1_ref[0].astype(jnp.bfloat16),
                    preferred_element_type=jnp.float32).astype(jnp.bfloat16) + b1
        h = jax.nn.gelu(h)
        acc = jnp.dot(h, w2_ref[0].astype(jnp.bfloat16),
                      preferred_element_type=jnp.float32)
        out_ref[rows, :] = out_ref[rows, :] + wcol * acc


def kernel(x, Wg, bg, Wr, br, W1, b1, W2, b2):
    xb = x.astype(jnp.bfloat16)
    W1r = W1.reshape(GE, D, F)
    W2r = W2.reshape(GE, F, D)
    b1r = b1.reshape(GE, 1, F)
    b2all = b2.reshape(GE, D)
    bg2 = bg.reshape(1, G)

    return pl.pallas_call(
        _moe_body,
        grid=(GE, NF),
        in_specs=[
            pl.BlockSpec((N, D), lambda j, f: (0, 0)),
            pl.BlockSpec((N, D), lambda j, f: (0, 0)),
            pl.BlockSpec((D, G), lambda j, f: (0, 0)),
            pl.BlockSpec((1, G), lambda j, f: (0, 0)),
            pl.BlockSpec((G, D, E), lambda j, f: (0, 0, 0)),
            pl.BlockSpec((G, E), lambda j, f: (0, 0)),
            pl.BlockSpec((1, D, F2), lambda j, f: (j, 0, f)),
            pl.BlockSpec((1, 1, F2), lambda j, f: (j, 0, f)),
            pl.BlockSpec((1, F2, D), lambda j, f: (j, f, 0)),
            pl.BlockSpec((GE, D), lambda j, f: (0, 0)),
        ],
        out_specs=pl.BlockSpec((N, D), lambda j, f: (0, 0)),
        out_shape=jax.ShapeDtypeStruct((N, D), jnp.float32),
        scratch_shapes=[pltpu.VMEM((N, GE), jnp.float32)],
        compiler_params=pltpu.CompilerParams(
            dimension_semantics=("arbitrary", "arbitrary"),
        ),
    )(x, xb, Wg, bg2, Wr, br, W1r, b1r, W2r, b2all)


# speedup vs baseline: 1.0449x; 1.0193x over previous
import jax
import jax.numpy as jnp
from jax.experimental import pallas as pl
from jax.experimental.pallas import tpu as pltpu

G = 3
E = 8
GE = G * E
D = 768
F = 3072
N = 2048
TEMP = 1.0

TN = 512
NT = N // TN


def _moe_body(xb_ref, Wg_ref, bg_ref, Wr_ref, br_ref,
              w1_ref, b1_ref, w2_ref, b2all_ref, out_ref, w_sc):
    j = pl.program_id(0)

    @pl.when(j == 0)
    def _router_and_init():
        xb = xb_ref[...]
        gl = jnp.dot(xb, Wg_ref[...].astype(jnp.bfloat16),
                     preferred_element_type=jnp.float32)
        gl = (gl + bg_ref[...]) / TEMP
        gl = gl - jnp.max(gl, axis=-1, keepdims=True)
        gex = jnp.exp(gl)
        gp = gex / jnp.sum(gex, axis=-1, keepdims=True)
        parts = []
        for g in range(G):
            el = jnp.dot(xb, Wr_ref[g].astype(jnp.bfloat16),
                         preferred_element_type=jnp.float32)
            el = (el + br_ref[g:g + 1, :]) / TEMP
            el = el - jnp.max(el, axis=-1, keepdims=True)
            eex = jnp.exp(el)
            ep = eex / jnp.sum(eex, axis=-1, keepdims=True)
            parts.append(ep * gp[:, g:g + 1])
        w = jnp.concatenate(parts, axis=-1)
        w_sc[...] = w.astype(jnp.bfloat16)
        out_ref[...] = xb.astype(jnp.float32) + jnp.dot(
            w.astype(jnp.bfloat16), b2all_ref[...].astype(jnp.bfloat16),
            preferred_element_type=jnp.float32)

    b1 = b1_ref[0].astype(jnp.bfloat16)
    lane = jax.lax.broadcasted_iota(jnp.int32, (1, GE), 1)
    for t in range(NT):
        rows = pl.ds(t * TN, TN)
        wcol = jnp.sum(jnp.where(lane == j,
                                 w_sc[rows, :].astype(jnp.float32), 0.0),
                       axis=-1, keepdims=True)
        xt = xb_ref[rows, :]
        h = jnp.dot(xt, w1_ref[0].astype(jnp.bfloat16),
                    preferred_element_type=jnp.float32).astype(jnp.bfloat16) + b1
        h = jax.nn.gelu(h)
        acc = jnp.dot(h, w2_ref[0].astype(jnp.bfloat16),
                      preferred_element_type=jnp.float32)
        out_ref[rows, :] = out_ref[rows, :] + wcol * acc


def kernel(x, Wg, bg, Wr, br, W1, b1, W2, b2):
    xb = x.astype(jnp.bfloat16)
    W1r = W1.reshape(GE, D, F)
    W2r = W2.reshape(GE, F, D)
    b1r = b1.reshape(GE, 1, F)
    b2all = b2.reshape(GE, D)
    bg2 = bg.reshape(1, G)

    return pl.pallas_call(
        _moe_body,
        grid=(GE,),
        in_specs=[
            pl.BlockSpec((N, D), lambda j: (0, 0)),
            pl.BlockSpec((D, G), lambda j: (0, 0)),
            pl.BlockSpec((1, G), lambda j: (0, 0)),
            pl.BlockSpec((G, D, E), lambda j: (0, 0, 0)),
            pl.BlockSpec((G, E), lambda j: (0, 0)),
            pl.BlockSpec((1, D, F), lambda j: (j, 0, 0)),
            pl.BlockSpec((1, 1, F), lambda j: (j, 0, 0)),
            pl.BlockSpec((1, F, D), lambda j: (j, 0, 0)),
            pl.BlockSpec((GE, D), lambda j: (0, 0)),
        ],
        out_specs=pl.BlockSpec((N, D), lambda j: (0, 0)),
        out_shape=jax.ShapeDtypeStruct((N, D), jnp.float32),
        scratch_shapes=[pltpu.VMEM((N, GE), jnp.bfloat16)],
        compiler_params=pltpu.CompilerParams(
            dimension_semantics=("arbitrary",),
            vmem_limit_bytes=64 * 1024 * 1024,
        ),
    )(xb, Wg, bg2, Wr, br, W1r, b1r, W2r, b2all)
